# baseline (speedup 1.0000x reference)
import functools

import jax
import jax.numpy as jnp
from jax import lax
from jax.experimental import pallas as pl
from jax.experimental.pallas import tpu as pltpu
from jax.experimental.pallas import tpu_sc as plsc

N = 10000
E = 320000
D = 128
DO = 64

NC = 2
NS = 16
NW = NC * NS
K = 128

N_PAD = 10112
RPT = N_PAD // NS
CHUNK = ((E + NW * K - 1) // (NW * K)) * K
STEPS = CHUNK // K
E_PAD = CHUNK * NW

_mesh = plsc.VectorSubcoreMesh(core_axis_name="c", subcore_axis_name="s")


@functools.partial(
    pl.kernel,
    out_type=jax.ShapeDtypeStruct((NC, N_PAD, D), jnp.float32),
    mesh=_mesh,
    scratch_types=[
        pltpu.VMEM((K,), jnp.int32),
        pltpu.VMEM((K, D), jnp.float32),
        pltpu.VMEM_SHARED((N_PAD, D), jnp.float32),
        pltpu.SemaphoreType.DMA,
    ],
)
def _deg_kernel(e0_hbm, z_hbm, col_hbm, out_hbm, col_v, ones_v, hist_sh, sem):
    cid = lax.axis_index("c")
    sid = lax.axis_index("s")
    r0 = sid * RPT
    pltpu.sync_copy(z_hbm.at[pl.ds(r0, RPT)], hist_sh.at[pl.ds(r0, RPT)])
    pltpu.sync_copy(e0_hbm, ones_v)
    plsc.subcore_barrier()
    base = (sid * NC + cid) * CHUNK

    def step(i, carry):
        pltpu.sync_copy(col_hbm.at[pl.ds(base + i * K, K)], col_v)
        pltpu.sync_copy(ones_v, hist_sh.at[col_v], add=True)
        return carry

    lax.fori_loop(0, STEPS, step, 0)
    plsc.subcore_barrier()
    pltpu.sync_copy(hist_sh.at[pl.ds(r0, RPT)], out_hbm.at[cid, pl.ds(r0, RPT)])


@functools.partial(
    pl.kernel,
    out_type=jax.ShapeDtypeStruct((NC, N_PAD, D), jnp.float32),
    mesh=_mesh,
    scratch_types=[
        pltpu.VMEM((K,), jnp.int32),
        pltpu.VMEM((K,), jnp.int32),
        pltpu.VMEM((K,), jnp.int32),
        pltpu.VMEM((K,), jnp.int32),
        pltpu.VMEM((K, D), jnp.float32),
        pltpu.VMEM((K, D), jnp.float32),
        pltpu.VMEM_SHARED((N_PAD, D), jnp.float32),
        pltpu.SemaphoreType.DMA,
        pltpu.SemaphoreType.DMA,
        pltpu.SemaphoreType.DMA,
        pltpu.SemaphoreType.DMA,
    ],
)
def _agg_kernel(hs_hbm, z_hbm, row_hbm, col_hbm, out_hbm, row_v0, row_v1,
                col_v0, col_v1, buf0, buf1, acc_sh, sem0, sem1, ssem0, ssem1):
    cid = lax.axis_index("c")
    sid = lax.axis_index("s")
    r0 = sid * RPT
    base = (sid * NC + cid) * CHUNK
    pltpu.sync_copy(z_hbm.at[pl.ds(r0, RPT)], acc_sh.at[pl.ds(r0, RPT)])
    plsc.subcore_barrier()

    pltpu.sync_copy(row_hbm.at[pl.ds(base, K)], row_v0)
    pltpu.async_copy(hs_hbm.at[row_v0], buf0, sem0)
    pltpu.sync_copy(row_hbm.at[pl.ds(base + K, K)], row_v1)
    pltpu.async_copy(hs_hbm.at[row_v1], buf1, sem1)

    def step(j, carry):
        i = 2 * j
        pltpu.make_async_copy(hs_hbm.at[row_v0], buf0, sem0).wait()
        pltpu.sync_copy(col_hbm.at[pl.ds(base + i * K, K)], col_v0)
        pltpu.async_copy(buf0, acc_sh.at[col_v0], ssem0, add=True)
        pltpu.make_async_copy(hs_hbm.at[row_v1], buf1, sem1).wait()
        pltpu.sync_copy(col_hbm.at[pl.ds(base + (i + 1) * K, K)], col_v1)
        pltpu.async_copy(buf1, acc_sh.at[col_v1], ssem1, add=True)
        pltpu.make_async_copy(buf0, acc_sh.at[col_v0], ssem0).wait()
        pltpu.sync_copy(row_hbm.at[pl.ds(base + (i + 2) * K, K)], row_v0)
        pltpu.async_copy(hs_hbm.at[row_v0], buf0, sem0)
        pltpu.make_async_copy(buf1, acc_sh.at[col_v1], ssem1).wait()
        pltpu.sync_copy(row_hbm.at[pl.ds(base + (i + 3) * K, K)], row_v1)
        pltpu.async_copy(hs_hbm.at[row_v1], buf1, sem1)
        return carry

    lax.fori_loop(0, STEPS // 2 - 1, step, 0)
    pltpu.make_async_copy(hs_hbm.at[row_v0], buf0, sem0).wait()
    pltpu.sync_copy(col_hbm.at[pl.ds(base + (STEPS - 3) * K, K)], col_v0)
    pltpu.async_copy(buf0, acc_sh.at[col_v0], ssem0, add=True)
    pltpu.make_async_copy(hs_hbm.at[row_v1], buf1, sem1).wait()
    pltpu.sync_copy(col_hbm.at[pl.ds(base + (STEPS - 2) * K, K)], col_v1)
    pltpu.async_copy(buf1, acc_sh.at[col_v1], ssem1, add=True)
    pltpu.make_async_copy(buf0, acc_sh.at[col_v0], ssem0).wait()
    pltpu.sync_copy(row_hbm.at[pl.ds(base + (STEPS - 1) * K, K)], row_v0)
    pltpu.async_copy(hs_hbm.at[row_v0], buf0, sem0)
    pltpu.make_async_copy(hs_hbm.at[row_v0], buf0, sem0).wait()
    pltpu.sync_copy(col_hbm.at[pl.ds(base + (STEPS - 1) * K, K)], col_v0)
    pltpu.async_copy(buf0, acc_sh.at[col_v0], ssem0, add=True)
    pltpu.make_async_copy(buf0, acc_sh.at[col_v0], ssem0).wait()
    pltpu.make_async_copy(buf1, acc_sh.at[col_v1], ssem1).wait()
    plsc.subcore_barrier()
    pltpu.sync_copy(acc_sh.at[pl.ds(r0, RPT)], out_hbm.at[cid, pl.ds(r0, RPT)])


BN = 632
GRID = N_PAD // BN


def _s_of(dref):
    deg = dref[0, :, 0:1] + dref[1, :, 0:1] + 1.0
    return lax.rsqrt(deg)


def _layer_a_body(d_ref, x_ref, w_ref, o_ref):
    s = _s_of(d_ref)
    o_ref[...] = s * jnp.dot(x_ref[...], w_ref[...],
                             preferred_element_type=jnp.float32)


def _layer_b_body(d_ref, p_ref, hs_ref, b_ref, w_ref, o_ref):
    s = _s_of(d_ref)
    t = jnp.maximum(s * (p_ref[0] + p_ref[1] + hs_ref[...]) + b_ref[...], 0.0)
    o_ref[...] = s * jnp.dot(t, w_ref[...],
                             preferred_element_type=jnp.float32)


def _layer_c_body(d_ref, p_ref, hs_ref, b_ref, w_ref, bo_ref, o_ref):
    s = _s_of(d_ref)
    t = jnp.maximum(s * (p_ref[0] + p_ref[1] + hs_ref[...]) + b_ref[...], 0.0)
    logits = jnp.dot(t, w_ref[...], preferred_element_type=jnp.float32)
    logits = logits + bo_ref[...]
    m = jnp.max(logits, axis=1, keepdims=True)
    lse = m + jnp.log(jnp.sum(jnp.exp(logits - m), axis=1, keepdims=True))
    o_ref[...] = logits - lse


_d_spec = pl.BlockSpec((2, BN, D), lambda i: (0, i, 0))
_p_spec = pl.BlockSpec((2, BN, D), lambda i: (0, i, 0))
_row_spec = pl.BlockSpec((BN, D), lambda i: (i, 0))
_w_spec = pl.BlockSpec((D, D), lambda i: (0, 0))
_wo_spec = pl.BlockSpec((D, DO), lambda i: (0, 0))
_b_spec = pl.BlockSpec((1, D), lambda i: (0, 0))
_bo_spec = pl.BlockSpec((1, DO), lambda i: (0, 0))

_layer_a = pl.pallas_call(
    _layer_a_body,
    grid=(GRID,),
    in_specs=[_d_spec, _row_spec, _w_spec],
    out_specs=_row_spec,
    out_shape=jax.ShapeDtypeStruct((N_PAD, D), jnp.float32),
)

_layer_b = pl.pallas_call(
    _layer_b_body,
    grid=(GRID,),
    in_specs=[_d_spec, _p_spec, _row_spec, _b_spec, _w_spec],
    out_specs=_row_spec,
    out_shape=jax.ShapeDtypeStruct((N_PAD, D), jnp.float32),
)

_layer_c = pl.pallas_call(
    _layer_c_body,
    grid=(GRID,),
    in_specs=[_d_spec, _p_spec, _row_spec, _b_spec, _wo_spec, _bo_spec],
    out_specs=pl.BlockSpec((BN, DO), lambda i: (i, 0)),
    out_shape=jax.ShapeDtypeStruct((N_PAD, DO), jnp.float32),
)


def kernel(x, edge_index, W1, b1, W2, b2, Wout, bout):
    row = edge_index[0].astype(jnp.int32)
    col = edge_index[1].astype(jnp.int32)
    pad = E_PAD - E
    row_p = jnp.concatenate([row, jnp.full((pad,), N, jnp.int32)])
    col_p = jnp.concatenate([col, jnp.full((pad,), N, jnp.int32)])
    x_pad = jnp.pad(x, ((0, N_PAD - N), (0, 0)))

    e0 = jnp.zeros((K, D), jnp.float32).at[:, 0].set(1.0)
    z_rows = jnp.zeros((N_PAD, D), jnp.float32)

    degp = _deg_kernel(e0, z_rows, col_p)
    hs1 = _layer_a(degp, x_pad, W1)
    p1 = _agg_kernel(hs1, z_rows, row_p, col_p)
    hs2 = _layer_b(degp, p1, hs1, b1.reshape(1, D), W2)
    p2 = _agg_kernel(hs2, z_rows, row_p, col_p)
    outp = _layer_c(degp, p2, hs2, b2.reshape(1, D), Wout, bout.reshape(1, DO))
    return outp[:N]


# speedup vs baseline: 1.0324x; 1.0324x over previous
import functools

import jax
import jax.numpy as jnp
from jax import lax
from jax.experimental import pallas as pl
from jax.experimental.pallas import tpu as pltpu
from jax.experimental.pallas import tpu_sc as plsc

N = 10000
E = 320000
D = 128
DO = 64

NC = 2
NS = 16
NW = NC * NS
K = 128

N_PAD = 10112
RPT = N_PAD // NS
CHUNK = ((E + NW * K - 1) // (NW * K)) * K
STEPS = CHUNK // K
E_PAD = CHUNK * NW

_mesh = plsc.VectorSubcoreMesh(core_axis_name="c", subcore_axis_name="s")


@functools.partial(
    pl.kernel,
    out_type=jax.ShapeDtypeStruct((NC, N_PAD, D), jnp.float32),
    mesh=_mesh,
    scratch_types=[
        pltpu.VMEM((K,), jnp.int32),
        pltpu.VMEM((K, D), jnp.float32),
        pltpu.VMEM_SHARED((N_PAD, D), jnp.float32),
        pltpu.SemaphoreType.DMA,
    ],
)
def _deg_kernel(e0_hbm, z_hbm, col_hbm, out_hbm, col_v, ones_v, hist_sh, sem):
    cid = lax.axis_index("c")
    sid = lax.axis_index("s")
    r0 = sid * RPT
    pltpu.sync_copy(z_hbm.at[pl.ds(r0, RPT)], hist_sh.at[pl.ds(r0, RPT)])
    pltpu.sync_copy(e0_hbm, ones_v)
    plsc.subcore_barrier()
    base = (sid * NC + cid) * CHUNK

    def step(i, carry):
        pltpu.sync_copy(col_hbm.at[pl.ds(base + i * K, K)], col_v)
        pltpu.sync_copy(ones_v, hist_sh.at[col_v], add=True)
        return carry

    lax.fori_loop(0, STEPS, step, 0)
    plsc.subcore_barrier()
    pltpu.sync_copy(hist_sh.at[pl.ds(r0, RPT)], out_hbm.at[cid, pl.ds(r0, RPT)])


@functools.partial(
    pl.kernel,
    out_type=jax.ShapeDtypeStruct((NC, N_PAD, D), jnp.float32),
    mesh=_mesh,
    scratch_types=[
        pltpu.VMEM((K,), jnp.int32),
        pltpu.VMEM((K,), jnp.int32),
        pltpu.VMEM((K,), jnp.int32),
        pltpu.VMEM((K,), jnp.int32),
        pltpu.VMEM((K,), jnp.int32),
        pltpu.VMEM((K,), jnp.int32),
        pltpu.VMEM((K, D), jnp.float32),
        pltpu.VMEM((K, D), jnp.float32),
        pltpu.VMEM((K, D), jnp.float32),
        pltpu.VMEM_SHARED((N_PAD, D), jnp.float32),
        pltpu.SemaphoreType.DMA,
        pltpu.SemaphoreType.DMA,
        pltpu.SemaphoreType.DMA,
        pltpu.SemaphoreType.DMA,
        pltpu.SemaphoreType.DMA,
        pltpu.SemaphoreType.DMA,
    ],
)
def _agg_kernel(hs_hbm, z_hbm, row_hbm, col_hbm, out_hbm, row_v0, row_v1,
                row_v2, col_v0, col_v1, col_v2, buf0, buf1, buf2, acc_sh,
                sem0, sem1, sem2, ssem0, ssem1, ssem2):
    cid = lax.axis_index("c")
    sid = lax.axis_index("s")
    r0 = sid * RPT
    base = (sid * NC + cid) * CHUNK
    pltpu.sync_copy(z_hbm.at[pl.ds(r0, RPT)], acc_sh.at[pl.ds(r0, RPT)])
    plsc.subcore_barrier()

    pltpu.sync_copy(row_hbm.at[pl.ds(base, K)], row_v0)
    pltpu.async_copy(hs_hbm.at[row_v0], buf0, sem0)
    pltpu.sync_copy(row_hbm.at[pl.ds(base + K, K)], row_v1)
    pltpu.async_copy(hs_hbm.at[row_v1], buf1, sem1)
    pltpu.sync_copy(row_hbm.at[pl.ds(base + 2 * K, K)], row_v2)
    pltpu.async_copy(hs_hbm.at[row_v2], buf2, sem2)

    def step(j, carry):
        i = 3 * j
        pltpu.make_async_copy(hs_hbm.at[row_v0], buf0, sem0).wait()
        pltpu.sync_copy(col_hbm.at[pl.ds(base + i * K, K)], col_v0)
        pltpu.async_copy(buf0, acc_sh.at[col_v0], ssem0, add=True)
        pltpu.make_async_copy(hs_hbm.at[row_v1], buf1, sem1).wait()
        pltpu.sync_copy(col_hbm.at[pl.ds(base + (i + 1) * K, K)], col_v1)
        pltpu.async_copy(buf1, acc_sh.at[col_v1], ssem1, add=True)
        pltpu.make_async_copy(hs_hbm.at[row_v2], buf2, sem2).wait()
        pltpu.sync_copy(col_hbm.at[pl.ds(base + (i + 2) * K, K)], col_v2)
        pltpu.async_copy(buf2, acc_sh.at[col_v2], ssem2, add=True)
        pltpu.make_async_copy(buf0, acc_sh.at[col_v0], ssem0).wait()
        pltpu.sync_copy(row_hbm.at[pl.ds(base + (i + 3) * K, K)], row_v0)
        pltpu.async_copy(hs_hbm.at[row_v0], buf0, sem0)
        pltpu.make_async_copy(buf1, acc_sh.at[col_v1], ssem1).wait()
        pltpu.sync_copy(row_hbm.at[pl.ds(base + (i + 4) * K, K)], row_v1)
        pltpu.async_copy(hs_hbm.at[row_v1], buf1, sem1)
        pltpu.make_async_copy(buf2, acc_sh.at[col_v2], ssem2).wait()
        pltpu.sync_copy(row_hbm.at[pl.ds(base + (i + 5) * K, K)], row_v2)
        pltpu.async_copy(hs_hbm.at[row_v2], buf2, sem2)
        return carry

    lax.fori_loop(0, STEPS // 3 - 1, step, 0)
    pltpu.make_async_copy(hs_hbm.at[row_v0], buf0, sem0).wait()
    pltpu.sync_copy(col_hbm.at[pl.ds(base + (STEPS - 4) * K, K)], col_v0)
    pltpu.async_copy(buf0, acc_sh.at[col_v0], ssem0, add=True)
    pltpu.make_async_copy(hs_hbm.at[row_v1], buf1, sem1).wait()
    pltpu.sync_copy(col_hbm.at[pl.ds(base + (STEPS - 3) * K, K)], col_v1)
    pltpu.async_copy(buf1, acc_sh.at[col_v1], ssem1, add=True)
    pltpu.make_async_copy(hs_hbm.at[row_v2], buf2, sem2).wait()
    pltpu.sync_copy(col_hbm.at[pl.ds(base + (STEPS - 2) * K, K)], col_v2)
    pltpu.async_copy(buf2, acc_sh.at[col_v2], ssem2, add=True)
    pltpu.make_async_copy(buf0, acc_sh.at[col_v0], ssem0).wait()
    pltpu.sync_copy(row_hbm.at[pl.ds(base + (STEPS - 1) * K, K)], row_v0)
    pltpu.async_copy(hs_hbm.at[row_v0], buf0, sem0)
    pltpu.make_async_copy(hs_hbm.at[row_v0], buf0, sem0).wait()
    pltpu.sync_copy(col_hbm.at[pl.ds(base + (STEPS - 1) * K, K)], col_v0)
    pltpu.async_copy(buf0, acc_sh.at[col_v0], ssem0, add=True)
    pltpu.make_async_copy(buf0, acc_sh.at[col_v0], ssem0).wait()
    pltpu.make_async_copy(buf1, acc_sh.at[col_v1], ssem1).wait()
    pltpu.make_async_copy(buf2, acc_sh.at[col_v2], ssem2).wait()
    plsc.subcore_barrier()
    pltpu.sync_copy(acc_sh.at[pl.ds(r0, RPT)], out_hbm.at[cid, pl.ds(r0, RPT)])


BN = 632
GRID = N_PAD // BN


def _s_of(dref):
    deg = dref[0, :, 0:1] + dref[1, :, 0:1] + 1.0
    return lax.rsqrt(deg)


def _layer_a_body(d_ref, x_ref, w_ref, o_ref):
    s = _s_of(d_ref)
    o_ref[...] = s * jnp.dot(x_ref[...], w_ref[...],
                             preferred_element_type=jnp.float32)


def _layer_b_body(d_ref, p_ref, hs_ref, b_ref, w_ref, o_ref):
    s = _s_of(d_ref)
    t = jnp.maximum(s * (p_ref[0] + p_ref[1] + hs_ref[...]) + b_ref[...], 0.0)
    o_ref[...] = s * jnp.dot(t, w_ref[...],
                             preferred_element_type=jnp.float32)


def _layer_c_body(d_ref, p_ref, hs_ref, b_ref, w_ref, bo_ref, o_ref):
    s = _s_of(d_ref)
    t = jnp.maximum(s * (p_ref[0] + p_ref[1] + hs_ref[...]) + b_ref[...], 0.0)
    logits = jnp.dot(t, w_ref[...], preferred_element_type=jnp.float32)
    logits = logits + bo_ref[...]
    m = jnp.max(logits, axis=1, keepdims=True)
    lse = m + jnp.log(jnp.sum(jnp.exp(logits - m), axis=1, keepdims=True))
    o_ref[...] = logits - lse


_d_spec = pl.BlockSpec((2, BN, D), lambda i: (0, i, 0))
_p_spec = pl.BlockSpec((2, BN, D), lambda i: (0, i, 0))
_row_spec = pl.BlockSpec((BN, D), lambda i: (i, 0))
_w_spec = pl.BlockSpec((D, D), lambda i: (0, 0))
_wo_spec = pl.BlockSpec((D, DO), lambda i: (0, 0))
_b_spec = pl.BlockSpec((1, D), lambda i: (0, 0))
_bo_spec = pl.BlockSpec((1, DO), lambda i: (0, 0))

_layer_a = pl.pallas_call(
    _layer_a_body,
    grid=(GRID,),
    in_specs=[_d_spec, _row_spec, _w_spec],
    out_specs=_row_spec,
    out_shape=jax.ShapeDtypeStruct((N_PAD, D), jnp.float32),
)

_layer_b = pl.pallas_call(
    _layer_b_body,
    grid=(GRID,),
    in_specs=[_d_spec, _p_spec, _row_spec, _b_spec, _w_spec],
    out_specs=_row_spec,
    out_shape=jax.ShapeDtypeStruct((N_PAD, D), jnp.float32),
)

_layer_c = pl.pallas_call(
    _layer_c_body,
    grid=(GRID,),
    in_specs=[_d_spec, _p_spec, _row_spec, _b_spec, _wo_spec, _bo_spec],
    out_specs=pl.BlockSpec((BN, DO), lambda i: (i, 0)),
    out_shape=jax.ShapeDtypeStruct((N_PAD, DO), jnp.float32),
)


def kernel(x, edge_index, W1, b1, W2, b2, Wout, bout):
    row = edge_index[0].astype(jnp.int32)
    col = edge_index[1].astype(jnp.int32)
    pad = E_PAD - E
    row_p = jnp.concatenate([row, jnp.full((pad,), N, jnp.int32)])
    col_p = jnp.concatenate([col, jnp.full((pad,), N, jnp.int32)])
    x_pad = jnp.pad(x, ((0, N_PAD - N), (0, 0)))

    e0 = jnp.zeros((K, D), jnp.float32).at[:, 0].set(1.0)
    z_rows = jnp.zeros((N_PAD, D), jnp.float32)

    degp = _deg_kernel(e0, z_rows, col_p)
    hs1 = _layer_a(degp, x_pad, W1)
    p1 = _agg_kernel(hs1, z_rows, row_p, col_p)
    hs2 = _layer_b(degp, p1, hs1, b1.reshape(1, D), W2)
    p2 = _agg_kernel(hs2, z_rows, row_p, col_p)
    outp = _layer_c(degp, p2, hs2, b2.reshape(1, D), Wout, bout.reshape(1, DO))
    return outp[:N]


# speedup vs baseline: 1.0790x; 1.0451x over previous
import functools

import jax
import jax.numpy as jnp
from jax import lax
from jax.experimental import pallas as pl
from jax.experimental.pallas import tpu as pltpu
from jax.experimental.pallas import tpu_sc as plsc

N = 10000
E = 320000
D = 128
DO = 64

NC = 2
NS = 16
NW = NC * NS
K = 128

N_PAD = 10112
RPT = N_PAD // NS
CHUNK = ((E + NW * K - 1) // (NW * K)) * K
STEPS = CHUNK // K
E_PAD = CHUNK * NW

_mesh = plsc.VectorSubcoreMesh(core_axis_name="c", subcore_axis_name="s")


@functools.partial(
    pl.kernel,
    out_type=jax.ShapeDtypeStruct((NC, N_PAD, D), jnp.float32),
    mesh=_mesh,
    scratch_types=[
        pltpu.VMEM((K,), jnp.int32),
        pltpu.VMEM((K,), jnp.int32),
        pltpu.VMEM((K, D), jnp.float32),
        pltpu.VMEM_SHARED((N_PAD, D), jnp.float32),
        pltpu.SemaphoreType.DMA,
        pltpu.SemaphoreType.DMA,
    ],
)
def _deg_kernel(e0_hbm, z_hbm, col_hbm, out_hbm, col_v0, col_v1, ones_v,
                hist_sh, csem0, csem1):
    cid = lax.axis_index("c")
    sid = lax.axis_index("s")
    r0 = sid * RPT
    pltpu.sync_copy(z_hbm.at[pl.ds(r0, RPT)], hist_sh.at[pl.ds(r0, RPT)])
    pltpu.sync_copy(e0_hbm, ones_v)
    plsc.subcore_barrier()
    base = (sid * NC + cid) * CHUNK

    pltpu.async_copy(col_hbm.at[pl.ds(base, K)], col_v0, csem0)
    pltpu.async_copy(col_hbm.at[pl.ds(base + K, K)], col_v1, csem1)

    def step(j, carry):
        i = 2 * j
        pltpu.make_async_copy(col_hbm.at[pl.ds(base + i * K, K)], col_v0,
                              csem0).wait()
        pltpu.sync_copy(ones_v, hist_sh.at[col_v0], add=True)
        pltpu.async_copy(col_hbm.at[pl.ds(base + (i + 2) * K, K)], col_v0,
                         csem0)
        pltpu.make_async_copy(col_hbm.at[pl.ds(base + (i + 1) * K, K)],
                              col_v1, csem1).wait()
        pltpu.sync_copy(ones_v, hist_sh.at[col_v1], add=True)
        pltpu.async_copy(col_hbm.at[pl.ds(base + (i + 3) * K, K)], col_v1,
                         csem1)
        return carry

    lax.fori_loop(0, STEPS // 2 - 1, step, 0)
    pltpu.make_async_copy(col_hbm.at[pl.ds(base + (STEPS - 3) * K, K)],
                          col_v0, csem0).wait()
    pltpu.sync_copy(ones_v, hist_sh.at[col_v0], add=True)
    pltpu.async_copy(col_hbm.at[pl.ds(base + (STEPS - 1) * K, K)], col_v0,
                     csem0)
    pltpu.make_async_copy(col_hbm.at[pl.ds(base + (STEPS - 2) * K, K)],
                          col_v1, csem1).wait()
    pltpu.sync_copy(ones_v, hist_sh.at[col_v1], add=True)
    pltpu.make_async_copy(col_hbm.at[pl.ds(base + (STEPS - 1) * K, K)],
                          col_v0, csem0).wait()
    pltpu.sync_copy(ones_v, hist_sh.at[col_v0], add=True)
    plsc.subcore_barrier()
    pltpu.sync_copy(hist_sh.at[pl.ds(r0, RPT)], out_hbm.at[cid, pl.ds(r0, RPT)])


@functools.partial(
    pl.kernel,
    out_type=jax.ShapeDtypeStruct((NC, N_PAD, D), jnp.float32),
    mesh=_mesh,
    scratch_types=[
        pltpu.VMEM((K,), jnp.int32),
        pltpu.VMEM((K,), jnp.int32),
        pltpu.VMEM((K,), jnp.int32),
        pltpu.VMEM((K,), jnp.int32),
        pltpu.VMEM((K, D), jnp.float32),
        pltpu.VMEM((K, D), jnp.float32),
        pltpu.VMEM_SHARED((N_PAD, D), jnp.float32),
        pltpu.SemaphoreType.DMA,
        pltpu.SemaphoreType.DMA,
    ],
)
def _agg_kernel(hs_hbm, z_hbm, row_hbm, col_hbm, out_hbm, row_v0, row_v1,
                col_v0, col_v1, buf0, buf1, acc_sh, sem0, sem1):
    cid = lax.axis_index("c")
    sid = lax.axis_index("s")
    r0 = sid * RPT
    base = (sid * NC + cid) * CHUNK
    pltpu.sync_copy(z_hbm.at[pl.ds(r0, RPT)], acc_sh.at[pl.ds(r0, RPT)])
    plsc.subcore_barrier()

    pltpu.sync_copy(row_hbm.at[pl.ds(base, K)], row_v0)
    pltpu.async_copy(hs_hbm.at[row_v0], buf0, sem0)
    pltpu.sync_copy(row_hbm.at[pl.ds(base + K, K)], row_v1)
    pltpu.async_copy(hs_hbm.at[row_v1], buf1, sem1)

    def step(j, carry):
        i = 2 * j
        pltpu.make_async_copy(hs_hbm.at[row_v0], buf0, sem0).wait()
        pltpu.sync_copy(col_hbm.at[pl.ds(base + i * K, K)], col_v0)
        pltpu.sync_copy(buf0, acc_sh.at[col_v0], add=True)
        pltpu.sync_copy(row_hbm.at[pl.ds(base + (i + 2) * K, K)], row_v0)
        pltpu.async_copy(hs_hbm.at[row_v0], buf0, sem0)
        pltpu.make_async_copy(hs_hbm.at[row_v1], buf1, sem1).wait()
        pltpu.sync_copy(col_hbm.at[pl.ds(base + (i + 1) * K, K)], col_v1)
        pltpu.sync_copy(buf1, acc_sh.at[col_v1], add=True)
        pltpu.sync_copy(row_hbm.at[pl.ds(base + (i + 3) * K, K)], row_v1)
        pltpu.async_copy(hs_hbm.at[row_v1], buf1, sem1)
        return carry

    lax.fori_loop(0, STEPS // 2 - 1, step, 0)
    pltpu.make_async_copy(hs_hbm.at[row_v0], buf0, sem0).wait()
    pltpu.sync_copy(col_hbm.at[pl.ds(base + (STEPS - 3) * K, K)], col_v0)
    pltpu.sync_copy(buf0, acc_sh.at[col_v0], add=True)
    pltpu.sync_copy(row_hbm.at[pl.ds(base + (STEPS - 1) * K, K)], row_v0)
    pltpu.async_copy(hs_hbm.at[row_v0], buf0, sem0)
    pltpu.make_async_copy(hs_hbm.at[row_v1], buf1, sem1).wait()
    pltpu.sync_copy(col_hbm.at[pl.ds(base + (STEPS - 2) * K, K)], col_v1)
    pltpu.sync_copy(buf1, acc_sh.at[col_v1], add=True)
    pltpu.make_async_copy(hs_hbm.at[row_v0], buf0, sem0).wait()
    pltpu.sync_copy(col_hbm.at[pl.ds(base + (STEPS - 1) * K, K)], col_v0)
    pltpu.sync_copy(buf0, acc_sh.at[col_v0], add=True)
    plsc.subcore_barrier()
    pltpu.sync_copy(acc_sh.at[pl.ds(r0, RPT)], out_hbm.at[cid, pl.ds(r0, RPT)])


BN = 632
GRID = N_PAD // BN


def _s_of(dref):
    deg = dref[0, :, 0:1] + dref[1, :, 0:1] + 1.0
    return lax.rsqrt(deg)


def _layer_a_body(d_ref, x_ref, w_ref, o_ref):
    s = _s_of(d_ref)
    o_ref[...] = s * jnp.dot(x_ref[...], w_ref[...],
                             preferred_element_type=jnp.float32)


def _layer_b_body(d_ref, p_ref, hs_ref, b_ref, w_ref, o_ref):
    s = _s_of(d_ref)
    t = jnp.maximum(s * (p_ref[0] + p_ref[1] + hs_ref[...]) + b_ref[...], 0.0)
    o_ref[...] = s * jnp.dot(t, w_ref[...],
                             preferred_element_type=jnp.float32)


def _layer_c_body(d_ref, p_ref, hs_ref, b_ref, w_ref, bo_ref, o_ref):
    s = _s_of(d_ref)
    t = jnp.maximum(s * (p_ref[0] + p_ref[1] + hs_ref[...]) + b_ref[...], 0.0)
    logits = jnp.dot(t, w_ref[...], preferred_element_type=jnp.float32)
    logits = logits + bo_ref[...]
    m = jnp.max(logits, axis=1, keepdims=True)
    lse = m + jnp.log(jnp.sum(jnp.exp(logits - m), axis=1, keepdims=True))
    o_ref[...] = logits - lse


_d_spec = pl.BlockSpec((2, BN, D), lambda i: (0, i, 0))
_p_spec = pl.BlockSpec((2, BN, D), lambda i: (0, i, 0))
_row_spec = pl.BlockSpec((BN, D), lambda i: (i, 0))
_w_spec = pl.BlockSpec((D, D), lambda i: (0, 0))
_wo_spec = pl.BlockSpec((D, DO), lambda i: (0, 0))
_b_spec = pl.BlockSpec((1, D), lambda i: (0, 0))
_bo_spec = pl.BlockSpec((1, DO), lambda i: (0, 0))

_layer_a = pl.pallas_call(
    _layer_a_body,
    grid=(GRID,),
    in_specs=[_d_spec, _row_spec, _w_spec],
    out_specs=_row_spec,
    out_shape=jax.ShapeDtypeStruct((N_PAD, D), jnp.float32),
)

_layer_b = pl.pallas_call(
    _layer_b_body,
    grid=(GRID,),
    in_specs=[_d_spec, _p_spec, _row_spec, _b_spec, _w_spec],
    out_specs=_row_spec,
    out_shape=jax.ShapeDtypeStruct((N_PAD, D), jnp.float32),
)

_layer_c = pl.pallas_call(
    _layer_c_body,
    grid=(GRID,),
    in_specs=[_d_spec, _p_spec, _row_spec, _b_spec, _wo_spec, _bo_spec],
    out_specs=pl.BlockSpec((BN, DO), lambda i: (i, 0)),
    out_shape=jax.ShapeDtypeStruct((N_PAD, DO), jnp.float32),
)


def kernel(x, edge_index, W1, b1, W2, b2, Wout, bout):
    row = edge_index[0].astype(jnp.int32)
    col = edge_index[1].astype(jnp.int32)
    pad = E_PAD - E
    row_p = jnp.concatenate([row, jnp.full((pad,), N, jnp.int32)])
    col_p = jnp.concatenate([col, jnp.full((pad,), N, jnp.int32)])
    x_pad = jnp.pad(x, ((0, N_PAD - N), (0, 0)))

    e0 = jnp.zeros((K, D), jnp.float32).at[:, 0].set(1.0)
    z_rows = jnp.zeros((N_PAD, D), jnp.float32)

    degp = _deg_kernel(e0, z_rows, col_p)
    hs1 = _layer_a(degp, x_pad, W1)
    p1 = _agg_kernel(hs1, z_rows, row_p, col_p)
    hs2 = _layer_b(degp, p1, hs1, b1.reshape(1, D), W2)
    p2 = _agg_kernel(hs2, z_rows, row_p, col_p)
    outp = _layer_c(degp, p2, hs2, b2.reshape(1, D), Wout, bout.reshape(1, DO))
    return outp[:N]


# speedup vs baseline: 1.1629x; 1.0777x over previous
import functools

import jax
import jax.numpy as jnp
from jax import lax
from jax.experimental import pallas as pl
from jax.experimental.pallas import tpu as pltpu
from jax.experimental.pallas import tpu_sc as plsc

N = 10000
E = 320000
D = 128
DO = 64

NC = 2
NS = 16
NW = NC * NS
K = 128

N_PAD = 10112
RPT = N_PAD // NS
CHUNK = ((E + NW * K - 1) // (NW * K)) * K
STEPS = CHUNK // K
E_PAD = CHUNK * NW

_mesh = plsc.VectorSubcoreMesh(core_axis_name="c", subcore_axis_name="s")


@functools.partial(
    pl.kernel,
    out_type=jax.ShapeDtypeStruct((NC, N_PAD, D), jnp.float32),
    mesh=_mesh,
    scratch_types=[
        pltpu.VMEM((K,), jnp.int32),
        pltpu.VMEM((K,), jnp.int32),
        pltpu.VMEM((K, D), jnp.float32),
        pltpu.VMEM_SHARED((N_PAD, D), jnp.float32),
        pltpu.SemaphoreType.DMA,
        pltpu.SemaphoreType.DMA,
    ],
)
def _deg_kernel(e0_hbm, z_hbm, col_hbm, out_hbm, col_v0, col_v1, ones_v,
                hist_sh, csem0, csem1):
    cid = lax.axis_index("c")
    sid = lax.axis_index("s")
    r0 = sid * RPT
    pltpu.sync_copy(z_hbm.at[pl.ds(r0, RPT)], hist_sh.at[pl.ds(r0, RPT)])
    pltpu.sync_copy(e0_hbm, ones_v)
    plsc.subcore_barrier()
    base = (sid * NC + cid) * CHUNK

    pltpu.async_copy(col_hbm.at[pl.ds(base, K)], col_v0, csem0)
    pltpu.async_copy(col_hbm.at[pl.ds(base + K, K)], col_v1, csem1)

    def step(j, carry):
        i = 2 * j
        pltpu.make_async_copy(col_hbm.at[pl.ds(base + i * K, K)], col_v0,
                              csem0).wait()
        pltpu.sync_copy(ones_v, hist_sh.at[col_v0], add=True)
        pltpu.async_copy(col_hbm.at[pl.ds(base + (i + 2) * K, K)], col_v0,
                         csem0)
        pltpu.make_async_copy(col_hbm.at[pl.ds(base + (i + 1) * K, K)],
                              col_v1, csem1).wait()
        pltpu.sync_copy(ones_v, hist_sh.at[col_v1], add=True)
        pltpu.async_copy(col_hbm.at[pl.ds(base + (i + 3) * K, K)], col_v1,
                         csem1)
        return carry

    lax.fori_loop(0, STEPS // 2 - 1, step, 0)
    pltpu.make_async_copy(col_hbm.at[pl.ds(base + (STEPS - 3) * K, K)],
                          col_v0, csem0).wait()
    pltpu.sync_copy(ones_v, hist_sh.at[col_v0], add=True)
    pltpu.async_copy(col_hbm.at[pl.ds(base + (STEPS - 1) * K, K)], col_v0,
                     csem0)
    pltpu.make_async_copy(col_hbm.at[pl.ds(base + (STEPS - 2) * K, K)],
                          col_v1, csem1).wait()
    pltpu.sync_copy(ones_v, hist_sh.at[col_v1], add=True)
    pltpu.make_async_copy(col_hbm.at[pl.ds(base + (STEPS - 1) * K, K)],
                          col_v0, csem0).wait()
    pltpu.sync_copy(ones_v, hist_sh.at[col_v0], add=True)
    plsc.subcore_barrier()
    pltpu.sync_copy(hist_sh.at[pl.ds(r0, RPT)], out_hbm.at[cid, pl.ds(r0, RPT)])


@functools.partial(
    pl.kernel,
    out_type=jax.ShapeDtypeStruct((NC, N_PAD, D), jnp.float32),
    mesh=_mesh,
    scratch_types=[
        pltpu.VMEM((K,), jnp.int32),
        pltpu.VMEM((K,), jnp.int32),
        pltpu.VMEM((K,), jnp.int32),
        pltpu.VMEM((K,), jnp.int32),
        pltpu.VMEM((K, D), jnp.float32),
        pltpu.VMEM((K, D), jnp.float32),
        pltpu.VMEM_SHARED((N_PAD, D), jnp.float32),
        pltpu.SemaphoreType.DMA,
        pltpu.SemaphoreType.DMA,
    ],
)
def _agg_kernel(hs_hbm, z_hbm, row_hbm, col_hbm, out_hbm, row_v0, row_v1,
                col_v0, col_v1, buf0, buf1, acc_sh, sem0, sem1):
    cid = lax.axis_index("c")
    sid = lax.axis_index("s")
    r0 = sid * RPT
    base = (sid * NC + cid) * CHUNK
    pltpu.sync_copy(z_hbm.at[pl.ds(r0, RPT)], acc_sh.at[pl.ds(r0, RPT)])
    plsc.subcore_barrier()

    hs_own = hs_hbm.at[cid]
    pltpu.sync_copy(row_hbm.at[pl.ds(base, K)], row_v0)
    pltpu.async_copy(hs_own.at[row_v0], buf0, sem0)
    pltpu.sync_copy(row_hbm.at[pl.ds(base + K, K)], row_v1)
    pltpu.async_copy(hs_own.at[row_v1], buf1, sem1)

    def step(j, carry):
        i = 2 * j
        pltpu.make_async_copy(hs_own.at[row_v0], buf0, sem0).wait()
        pltpu.sync_copy(col_hbm.at[pl.ds(base + i * K, K)], col_v0)
        pltpu.sync_copy(buf0, acc_sh.at[col_v0], add=True)
        pltpu.sync_copy(row_hbm.at[pl.ds(base + (i + 2) * K, K)], row_v0)
        pltpu.async_copy(hs_own.at[row_v0], buf0, sem0)
        pltpu.make_async_copy(hs_own.at[row_v1], buf1, sem1).wait()
        pltpu.sync_copy(col_hbm.at[pl.ds(base + (i + 1) * K, K)], col_v1)
        pltpu.sync_copy(buf1, acc_sh.at[col_v1], add=True)
        pltpu.sync_copy(row_hbm.at[pl.ds(base + (i + 3) * K, K)], row_v1)
        pltpu.async_copy(hs_own.at[row_v1], buf1, sem1)
        return carry

    lax.fori_loop(0, STEPS // 2 - 1, step, 0)
    pltpu.make_async_copy(hs_own.at[row_v0], buf0, sem0).wait()
    pltpu.sync_copy(col_hbm.at[pl.ds(base + (STEPS - 3) * K, K)], col_v0)
    pltpu.sync_copy(buf0, acc_sh.at[col_v0], add=True)
    pltpu.sync_copy(row_hbm.at[pl.ds(base + (STEPS - 1) * K, K)], row_v0)
    pltpu.async_copy(hs_own.at[row_v0], buf0, sem0)
    pltpu.make_async_copy(hs_own.at[row_v1], buf1, sem1).wait()
    pltpu.sync_copy(col_hbm.at[pl.ds(base + (STEPS - 2) * K, K)], col_v1)
    pltpu.sync_copy(buf1, acc_sh.at[col_v1], add=True)
    pltpu.make_async_copy(hs_own.at[row_v0], buf0, sem0).wait()
    pltpu.sync_copy(col_hbm.at[pl.ds(base + (STEPS - 1) * K, K)], col_v0)
    pltpu.sync_copy(buf0, acc_sh.at[col_v0], add=True)
    plsc.subcore_barrier()
    pltpu.sync_copy(acc_sh.at[pl.ds(r0, RPT)], out_hbm.at[cid, pl.ds(r0, RPT)])


BN = 632
GRID = N_PAD // BN


def _s_of(dref):
    deg = dref[0, :, 0:1] + dref[1, :, 0:1] + 1.0
    return lax.rsqrt(deg)


def _layer_a_body(d_ref, x_ref, w_ref, o_ref):
    s = _s_of(d_ref)
    v = s * jnp.dot(x_ref[...], w_ref[...],
                    preferred_element_type=jnp.float32)
    o_ref[...] = jnp.broadcast_to(v[None], (2,) + v.shape)


def _layer_b_body(d_ref, p_ref, hs_ref, b_ref, w_ref, o_ref):
    s = _s_of(d_ref)
    t = jnp.maximum(s * (p_ref[0] + p_ref[1] + hs_ref[0]) + b_ref[...], 0.0)
    v = s * jnp.dot(t, w_ref[...], preferred_element_type=jnp.float32)
    o_ref[...] = jnp.broadcast_to(v[None], (2,) + v.shape)


def _layer_c_body(d_ref, p_ref, hs_ref, b_ref, w_ref, bo_ref, o_ref):
    s = _s_of(d_ref)
    t = jnp.maximum(s * (p_ref[0] + p_ref[1] + hs_ref[0]) + b_ref[...], 0.0)
    logits = jnp.dot(t, w_ref[...], preferred_element_type=jnp.float32)
    logits = logits + bo_ref[...]
    m = jnp.max(logits, axis=1, keepdims=True)
    lse = m + jnp.log(jnp.sum(jnp.exp(logits - m), axis=1, keepdims=True))
    o_ref[...] = logits - lse


_d_spec = pl.BlockSpec((2, BN, D), lambda i: (0, i, 0))
_p_spec = pl.BlockSpec((2, BN, D), lambda i: (0, i, 0))
_row_spec = pl.BlockSpec((BN, D), lambda i: (i, 0))
_w_spec = pl.BlockSpec((D, D), lambda i: (0, 0))
_wo_spec = pl.BlockSpec((D, DO), lambda i: (0, 0))
_b_spec = pl.BlockSpec((1, D), lambda i: (0, 0))
_bo_spec = pl.BlockSpec((1, DO), lambda i: (0, 0))

_hs_spec = pl.BlockSpec((1, BN, D), lambda i: (0, i, 0))
_dup_spec = pl.BlockSpec((2, BN, D), lambda i: (0, i, 0))

_layer_a = pl.pallas_call(
    _layer_a_body,
    grid=(GRID,),
    in_specs=[_d_spec, _row_spec, _w_spec],
    out_specs=_dup_spec,
    out_shape=jax.ShapeDtypeStruct((2, N_PAD, D), jnp.float32),
)

_layer_b = pl.pallas_call(
    _layer_b_body,
    grid=(GRID,),
    in_specs=[_d_spec, _p_spec, _hs_spec, _b_spec, _w_spec],
    out_specs=_dup_spec,
    out_shape=jax.ShapeDtypeStruct((2, N_PAD, D), jnp.float32),
)

_layer_c = pl.pallas_call(
    _layer_c_body,
    grid=(GRID,),
    in_specs=[_d_spec, _p_spec, _hs_spec, _b_spec, _wo_spec, _bo_spec],
    out_specs=pl.BlockSpec((BN, DO), lambda i: (i, 0)),
    out_shape=jax.ShapeDtypeStruct((N_PAD, DO), jnp.float32),
)


def kernel(x, edge_index, W1, b1, W2, b2, Wout, bout):
    row = edge_index[0].astype(jnp.int32)
    col = edge_index[1].astype(jnp.int32)
    pad = E_PAD - E
    row_p = jnp.concatenate([row, jnp.full((pad,), N, jnp.int32)])
    col_p = jnp.concatenate([col, jnp.full((pad,), N, jnp.int32)])
    x_pad = jnp.pad(x, ((0, N_PAD - N), (0, 0)))

    e0 = jnp.zeros((K, D), jnp.float32).at[:, 0].set(1.0)
    z_rows = jnp.zeros((N_PAD, D), jnp.float32)

    degp = _deg_kernel(e0, z_rows, col_p)
    hs1 = _layer_a(degp, x_pad, W1)
    p1 = _agg_kernel(hs1, z_rows, row_p, col_p)
    hs2 = _layer_b(degp, p1, hs1, b1.reshape(1, D), W2)
    p2 = _agg_kernel(hs2, z_rows, row_p, col_p)
    outp = _layer_c(degp, p2, hs2, b2.reshape(1, D), Wout, bout.reshape(1, DO))
    return outp[:N]
